# baseline (speedup 1.0000x reference)
import functools

import jax
import jax.numpy as jnp
from jax import lax
from jax.experimental import pallas as pl
from jax.experimental.pallas import tpu as pltpu
from jax.experimental.pallas import tpu_sc as plsc

VOCAB_SIZE = 1000000
EMBED_DIM = 32
BATCH = 16384
N_FIELDS = 26

NUM_CORES = 2
NUM_SUBCORES = 16
NUM_WORKERS = NUM_CORES * NUM_SUBCORES

TOTAL_ROWS = BATCH * N_FIELDS
ROWS_PER_WORKER = TOTAL_ROWS // NUM_WORKERS
CHUNK = 832
N_BUF = 4
N_CHUNKS = ROWS_PER_WORKER // CHUNK

assert ROWS_PER_WORKER * NUM_WORKERS == TOTAL_ROWS
assert N_CHUNKS * CHUNK == ROWS_PER_WORKER and N_CHUNKS >= N_BUF

_mesh = plsc.VectorSubcoreMesh(
    core_axis_name="c", subcore_axis_name="s",
    num_cores=NUM_CORES, num_subcores=NUM_SUBCORES,
)


@functools.partial(
    pl.kernel,
    mesh=_mesh,
    compiler_params=pltpu.CompilerParams(use_tc_tiling_on_sc=False),
    out_type=jax.ShapeDtypeStruct((TOTAL_ROWS, EMBED_DIM), jnp.float32),
    scratch_types=[
        pltpu.VMEM((ROWS_PER_WORKER,), jnp.int32),
        pltpu.VMEM((CHUNK, EMBED_DIM), jnp.float32),
        pltpu.VMEM((CHUNK, EMBED_DIM), jnp.float32),
        pltpu.VMEM((CHUNK, EMBED_DIM), jnp.float32),
        pltpu.VMEM((CHUNK, EMBED_DIM), jnp.float32),
        pltpu.SemaphoreType.DMA,
        pltpu.SemaphoreType.DMA,
        pltpu.SemaphoreType.DMA,
        pltpu.SemaphoreType.DMA,
        pltpu.SemaphoreType.DMA,
        pltpu.SemaphoreType.DMA,
        pltpu.SemaphoreType.DMA,
        pltpu.SemaphoreType.DMA,
    ],
)
def _gather_kernel(idx_hbm, table_hbm, out_hbm, idx_v,
                   rows0, rows1, rows2, rows3,
                   gsem0, gsem1, gsem2, gsem3,
                   osem0, osem1, osem2, osem3):
    wid = lax.axis_index("s") * NUM_CORES + lax.axis_index("c")
    base = wid * ROWS_PER_WORKER

    rows = (rows0, rows1, rows2, rows3)
    gsems = (gsem0, gsem1, gsem2, gsem3)
    osems = (osem0, osem1, osem2, osem3)

    pltpu.sync_copy(idx_hbm.at[pl.ds(base, ROWS_PER_WORKER)], idx_v)

    def gather(g):
        b = g % N_BUF
        return pltpu.async_copy(
            table_hbm.at[idx_v.at[pl.ds(g * CHUNK, CHUNK)]], rows[b], gsems[b])

    gathers = [None] * N_CHUNKS
    writebacks = [None] * N_CHUNKS
    for g in range(N_BUF):
        gathers[g] = gather(g)
    for g in range(N_CHUNKS):
        b = g % N_BUF
        gathers[g].wait()
        writebacks[g] = pltpu.async_copy(
            rows[b], out_hbm.at[pl.ds(base + g * CHUNK, CHUNK)], osems[b])
        if g + N_BUF < N_CHUNKS:
            writebacks[g].wait()
            gathers[g + N_BUF] = gather(g + N_BUF)
    for g in range(N_CHUNKS - N_BUF, N_CHUNKS):
        writebacks[g].wait()


def kernel(ability_name, ability_embed_weight):
    flat_idx = ability_name.reshape(TOTAL_ROWS)
    out = _gather_kernel(flat_idx, ability_embed_weight)
    return out.reshape(BATCH, N_FIELDS, EMBED_DIM)


# speedup vs baseline: 1.0606x; 1.0606x over previous
import functools

import jax
import jax.numpy as jnp
from jax import lax
from jax.experimental import pallas as pl
from jax.experimental.pallas import tpu as pltpu
from jax.experimental.pallas import tpu_sc as plsc

VOCAB_SIZE = 1000000
EMBED_DIM = 32
BATCH = 16384
N_FIELDS = 26

NUM_CORES = 2
NUM_SUBCORES = 16
NUM_WORKERS = NUM_CORES * NUM_SUBCORES

TOTAL_ROWS = BATCH * N_FIELDS
ROWS_PER_WORKER = TOTAL_ROWS // NUM_WORKERS
CHUNK = 832
N_BUF = 4
N_CHUNKS = ROWS_PER_WORKER // CHUNK

assert ROWS_PER_WORKER * NUM_WORKERS == TOTAL_ROWS
assert N_CHUNKS * CHUNK == ROWS_PER_WORKER and N_CHUNKS >= N_BUF

_mesh = plsc.VectorSubcoreMesh(
    core_axis_name="c", subcore_axis_name="s",
    num_cores=NUM_CORES, num_subcores=NUM_SUBCORES,
)


@functools.partial(
    pl.kernel,
    mesh=_mesh,
    compiler_params=pltpu.CompilerParams(use_tc_tiling_on_sc=False),
    out_type=jax.ShapeDtypeStruct((TOTAL_ROWS, EMBED_DIM), jnp.float32),
    scratch_types=[
        pltpu.VMEM((ROWS_PER_WORKER,), jnp.int32),
        pltpu.VMEM((CHUNK, EMBED_DIM), jnp.float32),
        pltpu.VMEM((CHUNK, EMBED_DIM), jnp.float32),
        pltpu.VMEM((CHUNK, EMBED_DIM), jnp.float32),
        pltpu.VMEM((CHUNK, EMBED_DIM), jnp.float32),
        pltpu.SemaphoreType.DMA,
        pltpu.SemaphoreType.DMA,
        pltpu.SemaphoreType.DMA,
        pltpu.SemaphoreType.DMA,
        pltpu.SemaphoreType.DMA,
        pltpu.SemaphoreType.DMA,
        pltpu.SemaphoreType.DMA,
        pltpu.SemaphoreType.DMA,
    ],
)
def _gather_kernel(idx_hbm, table_hbm, out_hbm, idx_v,
                   rows0, rows1, rows2, rows3,
                   gsem0, gsem1, gsem2, gsem3,
                   osem0, osem1, osem2, osem3):
    wid = lax.axis_index("s") * NUM_CORES + lax.axis_index("c")
    base = wid * ROWS_PER_WORKER

    rows = (rows0, rows1, rows2, rows3)
    gsems = (gsem0, gsem1, gsem2, gsem3)
    osems = (osem0, osem1, osem2, osem3)

    pltpu.sync_copy(idx_hbm.at[pl.ds(base, ROWS_PER_WORKER)], idx_v)

    def gather(g):
        b = g % N_BUF
        return pltpu.async_copy(
            table_hbm.at[idx_v.at[pl.ds(g * CHUNK, CHUNK)]], rows[b], gsems[b])

    gathers = [None] * N_CHUNKS
    writebacks = [None] * N_CHUNKS
    for g in range(N_BUF):
        gathers[g] = gather(g)
    for g in range(N_CHUNKS):
        b = g % N_BUF
        gathers[g].wait()
        writebacks[g] = pltpu.async_copy(
            rows[b], out_hbm.at[pl.ds(base + g * CHUNK, CHUNK)], osems[b])
        if g + N_BUF < N_CHUNKS:
            writebacks[g].wait()
            gathers[g + N_BUF] = gather(g + N_BUF)
    for g in range(N_CHUNKS - N_BUF, N_CHUNKS):
        writebacks[g].wait()


def kernel(ability_name, ability_embed_weight):
    flat_idx = ability_name.T.reshape(TOTAL_ROWS)
    out = _gather_kernel(flat_idx, ability_embed_weight)
    return out.reshape(N_FIELDS, BATCH, EMBED_DIM).transpose(1, 0, 2)
